# baseline (speedup 1.0000x reference)
import jax
import jax.numpy as jnp
import numpy as np
from jax import lax
from jax.experimental import pallas as pl
from jax.experimental.pallas import tpu as pltpu

F32 = jnp.float32
I32 = jnp.int32
_BNS = np.float32(np.sqrt(np.float32(1.0 + 1e-5)))


def _cumsum_lanes(m, n):
    r = m
    sh = 1
    while sh < n:
        z = jnp.zeros(r.shape[:-1] + (sh,), r.dtype)
        r = r + jnp.concatenate([z, r[:, :-sh]], axis=1)
        sh *= 2
    return r


def _fps_into(xv, yv, zv, n, k, out_ref):
    b = xv.shape[0]
    iota_n = lax.broadcasted_iota(I32, (b, n), 1)
    iota_k = lax.broadcasted_iota(I32, (b, k), 1)
    out_ref[...] = jnp.zeros((3, b, k), F32)

    def body(step, carry):
        dist, far = carry
        oh = (iota_n == far).astype(F32)
        cx = jnp.sum(oh * xv, axis=1, keepdims=True)
        cy = jnp.sum(oh * yv, axis=1, keepdims=True)
        cz = jnp.sum(oh * zv, axis=1, keepdims=True)
        khot = (iota_k == step).astype(F32)
        out_ref[0] = out_ref[0] + cx * khot
        out_ref[1] = out_ref[1] + cy * khot
        out_ref[2] = out_ref[2] + cz * khot
        d = (xv - cx) ** 2 + (yv - cy) ** 2 + (zv - cz) ** 2
        dist = jnp.minimum(dist, d)
        far = jnp.argmax(dist, axis=1, keepdims=True).astype(I32)
        return dist, far

    lax.fori_loop(0, k, body,
                  (jnp.full((b, n), 1e10, F32), jnp.zeros((b, 1), I32)),
                  unroll=4)


def _sampling_body(xt_ref, x1_ref, x2_ref):
    xv, yv, zv = xt_ref[0], xt_ref[1], xt_ref[2]
    _fps_into(xv, yv, zv, xv.shape[1], 512, x1_ref)
    _fps_into(x1_ref[0], x1_ref[1], x1_ref[2], 512, 128, x2_ref)


def _sa_body(xt_ref, a_src_refs, c_ref, wxt_ref, layer_refs, out_ref,
             sel_ref, sv_ref, a_ref, g_ref, *, n, ns, sblk, r2, cw):
    xt = xt_ref[0]
    c = c_ref[0]
    d2 = ((c[:, 0:1] - xt[0:1]) ** 2
          + (c[:, 1:2] - xt[1:2]) ** 2
          + (c[:, 2:3] - xt[2:3]) ** 2)
    mask = d2 <= r2
    rank = _cumsum_lanes(mask.astype(I32), n)
    count = rank[:, n - 1:n]
    slotv = jnp.where(mask, rank, 0)

    a = None
    for src_ref, wt_ref in a_src_refs:
        term = jnp.dot(src_ref[0], wt_ref[...], preferred_element_type=F32)
        a = term if a is None else a + term
    a_ref[...] = a.astype(jnp.bfloat16)
    e = jnp.dot(c, wxt_ref[...], preferred_element_type=F32)

    cand = (slotv >= 1) & (slotv <= ns)
    iota_n = lax.broadcasted_iota(I32, (sblk, n), 1)
    last = jnp.max(jnp.where(cand, iota_n, 0))
    nchunks = last // cw + 1
    sv_ref[...] = slotv
    g_ref[...] = jnp.zeros_like(g_ref)

    def chunk(ci, _):
        sv = sv_ref[:, pl.ds(ci * cw, cw)]
        for j in range(ns):
            sel_ref[j * sblk:(j + 1) * sblk, :] = (
                (sv == (j + 1)).astype(jnp.bfloat16))
        g_ref[...] = g_ref[...] + jnp.dot(
            sel_ref[...], a_ref[pl.ds(ci * cw, cw), :],
            preferred_element_type=F32)
        return 0

    lax.fori_loop(0, nchunks, chunk, 0)
    g = g_ref[...]

    et = jnp.concatenate([e] * ns, axis=0)
    (b1, g1, e1), lay2, lay3 = layer_refs
    h = g - et + b1[...]
    h = jax.nn.relu(h / _BNS * g1[...] + e1[...])
    for (wt, bb, gg, be) in (lay2, lay3):
        h = jnp.dot(h, wt[...], preferred_element_type=F32) + bb[...]
        h = jax.nn.relu(h / _BNS * gg[...] + be[...])

    m = h[0:sblk]
    for j in range(1, ns):
        hj = h[j * sblk:(j + 1) * sblk]
        m = jnp.maximum(m, jnp.where(count > j, hj, 0.0))
    out_ref[0] = m


def _sa1_kernel_body(xt_ref, xyz_ref, c_ref, wxt_ref,
                     b1, g1, e1, w2t, b2, g2, e2, w3t, b3, g3, e3,
                     out_ref, sel_ref, sv_ref, a_ref, g_ref):
    _sa_body(xt_ref, [(xyz_ref, wxt_ref)], c_ref, wxt_ref,
             [(b1, g1, e1), (w2t, b2, g2, e2), (w3t, b3, g3, e3)],
             out_ref, sel_ref, sv_ref, a_ref, g_ref, n=2048, ns=32, sblk=64,
             r2=jnp.float32(0.2 ** 2), cw=256)


def _sa2_kernel_body(xt_ref, x1_ref, p1_ref, c_ref, wxt_ref, wft_ref,
                     b1, g1, e1, w2t, b2, g2, e2, w3t, b3, g3, e3,
                     out_ref, sel_ref, sv_ref, a_ref, g_ref):
    _sa_body(xt_ref, [(x1_ref, wxt_ref), (p1_ref, wft_ref)], c_ref, wxt_ref,
             [(b1, g1, e1), (w2t, b2, g2, e2), (w3t, b3, g3, e3)],
             out_ref, sel_ref, sv_ref, a_ref, g_ref, n=512, ns=64, sblk=64,
             r2=jnp.float32(0.4 ** 2), cw=128)


def _head_body(rows_ref, w1t, b1, g1, e1, w2t, b2, g2, e2, w3t, b3, g3, e3,
               w4t, b4, g4, e4, w5t, b5, g5, e5, w6t, b6,
               out_ref, pool_ref):
    h = rows_ref[...]
    for (wt, bb, gg, be) in ((w1t, b1, g1, e1), (w2t, b2, g2, e2),
                             (w3t, b3, g3, e3)):
        h = jnp.dot(h, wt[...], preferred_element_type=F32) + bb[...]
        h = jax.nn.relu(h / _BNS * gg[...] + be[...])
    nb = pool_ref.shape[0]
    for b in range(nb):
        pool_ref[b:b + 1, :] = jnp.max(h[b * 128:(b + 1) * 128], axis=0,
                                       keepdims=True)
    x = pool_ref[...]
    for (wt, bb, gg, be) in ((w4t, b4, g4, e4), (w5t, b5, g5, e5)):
        x = jnp.dot(x, wt[...], preferred_element_type=F32) + bb[...]
        x = jax.nn.relu(x / _BNS * gg[...] + be[...])
    out_ref[...] = jnp.dot(x, w6t[...], preferred_element_type=F32) + b6[...]


def _mlp_args(mlp, skip_first_w=True):
    out = []
    for i, p in enumerate(mlp):
        if not (skip_first_w and i == 0):
            out.append(jnp.transpose(p["W"]))
        out.extend([p["b"][None, :], p["gamma"][None, :], p["beta"][None, :]])
    return out


def _full_spec(x):
    return pl.BlockSpec(x.shape, lambda *_: (0,) * x.ndim)


def kernel(xyz, params):
    B, N, _ = xyz.shape
    f = F32
    xt_all = jnp.transpose(xyz, (2, 0, 1))

    x1t, x2t = pl.pallas_call(
        _sampling_body,
        out_shape=[jax.ShapeDtypeStruct((3, B, 512), f),
                   jax.ShapeDtypeStruct((3, B, 128), f)],
    )(xt_all)
    x1 = jnp.transpose(x1t, (1, 2, 0))
    x2 = jnp.transpose(x2t, (1, 2, 0))
    xtb = jnp.transpose(xyz, (0, 2, 1))
    x1tb = jnp.transpose(x1, (0, 2, 1))

    sa1 = params["sa1"]
    w1xt = jnp.transpose(sa1[0]["W"])
    sa1_rest = _mlp_args(sa1)
    grid1 = (B, 512 // 64)
    p1 = pl.pallas_call(
        _sa1_kernel_body,
        grid=grid1,
        in_specs=[
            pl.BlockSpec((1, 3, 2048), lambda b, s: (b, 0, 0)),
            pl.BlockSpec((1, 2048, 3), lambda b, s: (b, 0, 0)),
            pl.BlockSpec((1, 64, 3), lambda b, s: (b, s, 0)),
            _full_spec(w1xt),
        ] + [_full_spec(a) for a in sa1_rest],
        out_specs=pl.BlockSpec((1, 64, 128), lambda b, s: (b, s, 0)),
        out_shape=jax.ShapeDtypeStruct((B, 512, 128), f),
        scratch_shapes=[pltpu.VMEM((32 * 64, 256), jnp.bfloat16),
                        pltpu.VMEM((64, 2048), jnp.int32),
                        pltpu.VMEM((2048, 64), jnp.bfloat16),
                        pltpu.VMEM((32 * 64, 64), f)],
    )(xtb, xyz, x1, w1xt, *sa1_rest)

    sa2 = params["sa2"]
    w2xt = jnp.transpose(sa2[0]["W"][:, :3])
    w2ft = jnp.transpose(sa2[0]["W"][:, 3:])
    sa2_rest = _mlp_args(sa2)
    grid2 = (B, 128 // 64)
    p2 = pl.pallas_call(
        _sa2_kernel_body,
        grid=grid2,
        in_specs=[
            pl.BlockSpec((1, 3, 512), lambda b, s: (b, 0, 0)),
            pl.BlockSpec((1, 512, 3), lambda b, s: (b, 0, 0)),
            pl.BlockSpec((1, 512, 128), lambda b, s: (b, 0, 0)),
            pl.BlockSpec((1, 64, 3), lambda b, s: (b, s, 0)),
            _full_spec(w2xt),
            _full_spec(w2ft),
        ] + [_full_spec(a) for a in sa2_rest],
        out_specs=pl.BlockSpec((1, 64, 256), lambda b, s: (b, s, 0)),
        out_shape=jax.ShapeDtypeStruct((B, 128, 256), f),
        scratch_shapes=[pltpu.VMEM((64 * 64, 128), jnp.bfloat16),
                        pltpu.VMEM((64, 512), jnp.int32),
                        pltpu.VMEM((512, 128), jnp.bfloat16),
                        pltpu.VMEM((64 * 64, 128), f)],
    )(x1tb, x1, p1, x2, w2xt, w2ft, *sa2_rest)

    rows = jnp.concatenate([x2, p2], axis=-1).reshape(B * 128, 259)
    sa3 = params["sa3"]
    head_args = []
    for p in sa3:
        head_args.extend([jnp.transpose(p["W"]), p["b"][None, :],
                          p["gamma"][None, :], p["beta"][None, :]])
    for nm in ("fc4", "fc5"):
        p = params[nm]
        head_args.extend([jnp.transpose(p["W"]), p["b"][None, :],
                          p["gamma"][None, :], p["beta"][None, :]])
    head_args.extend([jnp.transpose(params["fc6"]["W"]),
                      params["fc6"]["b"][None, :]])

    out = pl.pallas_call(
        _head_body,
        out_shape=jax.ShapeDtypeStruct((B, 40), f),
        scratch_shapes=[pltpu.VMEM((B, 1024), f)],
    )(rows, *head_args)
    return out


# speedup vs baseline: 1.3216x; 1.3216x over previous
import jax
import jax.numpy as jnp
import numpy as np
from jax import lax
from jax.experimental import pallas as pl
from jax.experimental.pallas import tpu as pltpu

F32 = jnp.float32
I32 = jnp.int32
_BNS = np.float32(np.sqrt(np.float32(1.0 + 1e-5)))


def _cumsum_lanes(m, n):
    r = m
    sh = 1
    while sh < n:
        z = jnp.zeros(r.shape[:-1] + (sh,), r.dtype)
        r = r + jnp.concatenate([z, r[:, :-sh]], axis=1)
        sh *= 2
    return r


def _fps_into(xv, yv, zv, n, k, out_ref):
    b = xv.shape[0]
    iota_n = lax.broadcasted_iota(I32, (b, n), 1)
    iota_k = lax.broadcasted_iota(I32, (b, k), 1)
    out_ref[...] = jnp.zeros((3, b, k), F32)

    def body(step, carry):
        dist, far = carry
        oh = (iota_n == far).astype(F32)
        cx = jnp.sum(oh * xv, axis=1, keepdims=True)
        cy = jnp.sum(oh * yv, axis=1, keepdims=True)
        cz = jnp.sum(oh * zv, axis=1, keepdims=True)
        khot = (iota_k == step).astype(F32)
        out_ref[0] = out_ref[0] + cx * khot
        out_ref[1] = out_ref[1] + cy * khot
        out_ref[2] = out_ref[2] + cz * khot
        d = (xv - cx) ** 2 + (yv - cy) ** 2 + (zv - cz) ** 2
        dist = jnp.minimum(dist, d)
        far = jnp.argmax(dist, axis=1, keepdims=True).astype(I32)
        return dist, far

    lax.fori_loop(0, k, body,
                  (jnp.full((b, n), 1e10, F32), jnp.zeros((b, 1), I32)),
                  unroll=8)


def _sampling_body(xt_ref, x1_ref, x2_ref):
    xv, yv, zv = xt_ref[0], xt_ref[1], xt_ref[2]
    _fps_into(xv, yv, zv, xv.shape[1], 512, x1_ref)
    _fps_into(x1_ref[0], x1_ref[1], x1_ref[2], 512, 128, x2_ref)


def _sa_body(xt_ref, a_src_refs, c_ref, wxt_ref, layer_refs, out_ref,
             sel_ref, *, n, ns, sblk, r2):
    xt = xt_ref[0]
    c = c_ref[0]
    d2 = ((c[:, 0:1] - xt[0:1]) ** 2
          + (c[:, 1:2] - xt[1:2]) ** 2
          + (c[:, 2:3] - xt[2:3]) ** 2)
    mask = d2 <= r2
    rank = _cumsum_lanes(mask.astype(I32), n)
    count = rank[:, n - 1:n]
    slotv = jnp.where(mask, rank, 0)

    a = None
    for src_ref, wt_ref in a_src_refs:
        term = jnp.dot(src_ref[0], wt_ref[...], preferred_element_type=F32)
        a = term if a is None else a + term
    a = a.astype(jnp.bfloat16)
    e = jnp.dot(c, wxt_ref[...], preferred_element_type=F32)

    for j in range(ns):
        sel_ref[j * sblk:(j + 1) * sblk, :] = (
            (slotv == (j + 1)).astype(jnp.bfloat16))
    g = jnp.dot(sel_ref[...], a, preferred_element_type=F32)

    et = jnp.concatenate([e] * ns, axis=0)
    (b1, g1, e1), lay2, lay3 = layer_refs
    h = g - et + b1[...]
    h = jax.nn.relu(h / _BNS * g1[...] + e1[...])
    for (wt, bb, gg, be) in (lay2, lay3):
        h = jnp.dot(h, wt[...], preferred_element_type=F32) + bb[...]
        h = jax.nn.relu(h / _BNS * gg[...] + be[...])

    m = h[0:sblk]
    for j in range(1, ns):
        hj = h[j * sblk:(j + 1) * sblk]
        m = jnp.maximum(m, jnp.where(count > j, hj, 0.0))
    out_ref[0] = m


def _sa1_kernel_body(xt_ref, xyz_ref, c_ref, wxt_ref,
                     b1, g1, e1, w2t, b2, g2, e2, w3t, b3, g3, e3,
                     out_ref, sel_ref):
    _sa_body(xt_ref, [(xyz_ref, wxt_ref)], c_ref, wxt_ref,
             [(b1, g1, e1), (w2t, b2, g2, e2), (w3t, b3, g3, e3)],
             out_ref, sel_ref, n=2048, ns=32, sblk=64,
             r2=jnp.float32(0.2 ** 2))


def _sa2_kernel_body(xt_ref, x1_ref, p1_ref, c_ref, wxt_ref, wft_ref,
                     b1, g1, e1, w2t, b2, g2, e2, w3t, b3, g3, e3,
                     out_ref, sel_ref):
    _sa_body(xt_ref, [(x1_ref, wxt_ref), (p1_ref, wft_ref)], c_ref, wxt_ref,
             [(b1, g1, e1), (w2t, b2, g2, e2), (w3t, b3, g3, e3)],
             out_ref, sel_ref, n=512, ns=64, sblk=64,
             r2=jnp.float32(0.4 ** 2))


def _head_body(rows_ref, w1t, b1, g1, e1, w2t, b2, g2, e2, w3t, b3, g3, e3,
               w4t, b4, g4, e4, w5t, b5, g5, e5, w6t, b6,
               out_ref, pool_ref):
    h = rows_ref[...]
    for (wt, bb, gg, be) in ((w1t, b1, g1, e1), (w2t, b2, g2, e2),
                             (w3t, b3, g3, e3)):
        h = jnp.dot(h, wt[...], preferred_element_type=F32) + bb[...]
        h = jax.nn.relu(h / _BNS * gg[...] + be[...])
    nb = pool_ref.shape[0]
    for b in range(nb):
        pool_ref[b:b + 1, :] = jnp.max(h[b * 128:(b + 1) * 128], axis=0,
                                       keepdims=True)
    x = pool_ref[...]
    for (wt, bb, gg, be) in ((w4t, b4, g4, e4), (w5t, b5, g5, e5)):
        x = jnp.dot(x, wt[...], preferred_element_type=F32) + bb[...]
        x = jax.nn.relu(x / _BNS * gg[...] + be[...])
    out_ref[...] = jnp.dot(x, w6t[...], preferred_element_type=F32) + b6[...]


def _mlp_args(mlp, skip_first_w=True):
    out = []
    for i, p in enumerate(mlp):
        if not (skip_first_w and i == 0):
            out.append(jnp.transpose(p["W"]))
        out.extend([p["b"][None, :], p["gamma"][None, :], p["beta"][None, :]])
    return out


def _full_spec(x):
    return pl.BlockSpec(x.shape, lambda *_: (0,) * x.ndim)


def kernel(xyz, params):
    B, N, _ = xyz.shape
    f = F32
    xt_all = jnp.transpose(xyz, (2, 0, 1))

    x1t, x2t = pl.pallas_call(
        _sampling_body,
        out_shape=[jax.ShapeDtypeStruct((3, B, 512), f),
                   jax.ShapeDtypeStruct((3, B, 128), f)],
    )(xt_all)
    x1 = jnp.transpose(x1t, (1, 2, 0))
    x2 = jnp.transpose(x2t, (1, 2, 0))
    xtb = jnp.transpose(xyz, (0, 2, 1))
    x1tb = jnp.transpose(x1, (0, 2, 1))

    sa1 = params["sa1"]
    w1xt = jnp.transpose(sa1[0]["W"])
    sa1_rest = _mlp_args(sa1)
    grid1 = (B, 512 // 64)
    p1 = pl.pallas_call(
        _sa1_kernel_body,
        grid=grid1,
        in_specs=[
            pl.BlockSpec((1, 3, 2048), lambda b, s: (b, 0, 0)),
            pl.BlockSpec((1, 2048, 3), lambda b, s: (b, 0, 0)),
            pl.BlockSpec((1, 64, 3), lambda b, s: (b, s, 0)),
            _full_spec(w1xt),
        ] + [_full_spec(a) for a in sa1_rest],
        out_specs=pl.BlockSpec((1, 64, 128), lambda b, s: (b, s, 0)),
        out_shape=jax.ShapeDtypeStruct((B, 512, 128), f),
        scratch_shapes=[pltpu.VMEM((32 * 64, 2048), jnp.bfloat16)],
    )(xtb, xyz, x1, w1xt, *sa1_rest)

    sa2 = params["sa2"]
    w2xt = jnp.transpose(sa2[0]["W"][:, :3])
    w2ft = jnp.transpose(sa2[0]["W"][:, 3:])
    sa2_rest = _mlp_args(sa2)
    grid2 = (B, 128 // 64)
    p2 = pl.pallas_call(
        _sa2_kernel_body,
        grid=grid2,
        in_specs=[
            pl.BlockSpec((1, 3, 512), lambda b, s: (b, 0, 0)),
            pl.BlockSpec((1, 512, 3), lambda b, s: (b, 0, 0)),
            pl.BlockSpec((1, 512, 128), lambda b, s: (b, 0, 0)),
            pl.BlockSpec((1, 64, 3), lambda b, s: (b, s, 0)),
            _full_spec(w2xt),
            _full_spec(w2ft),
        ] + [_full_spec(a) for a in sa2_rest],
        out_specs=pl.BlockSpec((1, 64, 256), lambda b, s: (b, s, 0)),
        out_shape=jax.ShapeDtypeStruct((B, 128, 256), f),
        scratch_shapes=[pltpu.VMEM((64 * 64, 512), jnp.bfloat16)],
    )(x1tb, x1, p1, x2, w2xt, w2ft, *sa2_rest)

    rows = jnp.concatenate([x2, p2], axis=-1).reshape(B * 128, 259)
    sa3 = params["sa3"]
    head_args = []
    for p in sa3:
        head_args.extend([jnp.transpose(p["W"]), p["b"][None, :],
                          p["gamma"][None, :], p["beta"][None, :]])
    for nm in ("fc4", "fc5"):
        p = params[nm]
        head_args.extend([jnp.transpose(p["W"]), p["b"][None, :],
                          p["gamma"][None, :], p["beta"][None, :]])
    head_args.extend([jnp.transpose(params["fc6"]["W"]),
                      params["fc6"]["b"][None, :]])

    out = pl.pallas_call(
        _head_body,
        out_shape=jax.ShapeDtypeStruct((B, 40), f),
        scratch_shapes=[pltpu.VMEM((B, 1024), f)],
    )(rows, *head_args)
    return out
